# baseline (speedup 1.0000x reference)
import jax
import jax.numpy as jnp
from jax import lax
from jax.experimental import pallas as pl
from jax.experimental.pallas import tpu as pltpu
from jax.experimental.pallas import tpu_sc as plsc

N = 50000
NPAD = 53248
E = 800000
EPAD = 819200
GRP = 128
NGRP = EPAD // GRP
NW = 32
GPW = NGRP // NW
IC = 20
NBLK = GPW // IC
NCH = IC
ROWS_PT = NPAD // 16
NZDMA = 8
ZR = 416
GACC = 1056
PGRP = 128
PG = NPAD // PGRP
PGPW = PG // NW
RB = 1024
GRIDN = NPAD // RB
F32 = jnp.float32


def _sc_agg(table, src2d, dst2d, F):
    mesh = plsc.VectorSubcoreMesh(core_axis_name="c", subcore_axis_name="s")
    fl = F // 16

    def body(table_ref, src_ref, dst_ref, out_ref, sidx, didx, rows,
             acc, sg0, sg1, ss0, ss1):
        c = lax.axis_index("c")
        s = lax.axis_index("s")
        w = c * 16 + s
        zv = jnp.zeros((16,), F32)
        semg = (sg0, sg1)
        sems = (ss0, ss1)

        def zinit(i, _):
            rows[i // fl, pl.ds((i % fl) * 16, 16)] = zv
            return 0
        lax.fori_loop(0, 2 * GRP * fl, zinit, 0)

        def zdma(q, _):
            pltpu.sync_copy(rows.at[pl.ds(0, ZR // 2)],
                            acc.at[pl.ds(s * ROWS_PT + q * (ZR // 2),
                                         ZR // 2)])
            return 0
        lax.fori_loop(0, 2 * NZDMA, zdma, 0)
        plsc.subcore_barrier()

        def gath(pc):
            par = pc % 2
            return pltpu.async_copy(
                table_ref.at[sidx.at[pc]],
                rows.at[pl.ds(par * GRP, GRP)], semg[par])

        def scat(pc):
            par = pc % 2
            return pltpu.async_copy(
                rows.at[pl.ds(par * GRP, GRP)],
                acc.at[didx.at[pc]], sems[par], add=True)

        def block(ib, _):
            g0 = w * GPW + ib * IC
            pltpu.sync_copy(src_ref.at[pl.ds(g0, IC)], sidx)
            pltpu.sync_copy(dst_ref.at[pl.ds(g0, IC)], didx)
            gd = {0: gath(0)}
            sd = {}
            for pc in range(NCH):
                gd.pop(pc).wait()
                if pc >= 1:
                    sd.pop(pc - 1).wait()
                if pc < NCH - 1:
                    gd[pc + 1] = gath(pc + 1)
                sd[pc] = scat(pc)
            sd.pop(NCH - 1).wait()
            return 0
        lax.fori_loop(0, NBLK, block, 0)
        plsc.subcore_barrier()
        pltpu.sync_copy(acc.at[pl.ds(s * ROWS_PT, ROWS_PT)],
                        out_ref.at[c, pl.ds(s * ROWS_PT, ROWS_PT)])

    k = pl.kernel(
        body,
        out_type=jax.ShapeDtypeStruct((2, NPAD, F), F32),
        mesh=mesh,
        compiler_params=pltpu.CompilerParams(use_tc_tiling_on_sc=False),
        scratch_types=[
            pltpu.VMEM((IC, GRP), jnp.int32),
            pltpu.VMEM((IC, GRP), jnp.int32),
            pltpu.VMEM((2 * GRP, F), F32),
            pltpu.VMEM_SHARED((NPAD, F), F32),
            pltpu.SemaphoreType.DMA,
            pltpu.SemaphoreType.DMA,
            pltpu.SemaphoreType.DMA,
            pltpu.SemaphoreType.DMA,
        ],
    )
    return k(table, src2d, dst2d)


def _sc_pool(h3, batch2d):
    mesh = plsc.VectorSubcoreMesh(core_axis_name="c", subcore_axis_name="s")
    GR = GACC // 16

    def body(h_ref, b_ref, outs_ref, outc_ref, rows, bidx, ones_v, zbs, zbc,
             accs, accc):
        c = lax.axis_index("c")
        s = lax.axis_index("s")
        w = c * 16 + s
        zv = jnp.zeros((16,), F32)
        ov = jnp.full((16,), 1.0, F32)

        def oinit(i, _):
            ones_v[i, pl.ds(0, 16)] = ov
            return 0
        lax.fori_loop(0, PGRP, oinit, 0)

        def zsinit(i, _):
            zbs[i // 4, pl.ds((i % 4) * 16, 16)] = zv
            return 0
        lax.fori_loop(0, GR * 4, zsinit, 0)

        def zcinit(i, _):
            zbc[i, pl.ds(0, 16)] = zv
            return 0
        lax.fori_loop(0, GR, zcinit, 0)
        pltpu.sync_copy(zbs, accs.at[pl.ds(s * GR, GR)])
        pltpu.sync_copy(zbc, accc.at[pl.ds(s * GR, GR)])
        plsc.subcore_barrier()

        def grp(k, _):
            g = w * PGPW + k
            pltpu.sync_copy(b_ref.at[pl.ds(g, 1)], bidx)
            pltpu.sync_copy(h_ref.at[pl.ds(g * PGRP, PGRP)], rows)
            pltpu.sync_copy(rows, accs.at[bidx.at[0]], add=True)
            pltpu.sync_copy(ones_v, accc.at[bidx.at[0]], add=True)
            return 0
        lax.fori_loop(0, PGPW, grp, 0)
        plsc.subcore_barrier()

        @pl.when(s == 0)
        def _():
            pltpu.sync_copy(accs, outs_ref.at[c])
            pltpu.sync_copy(accc, outc_ref.at[c])

    k = pl.kernel(
        body,
        out_type=(jax.ShapeDtypeStruct((2, GACC, 64), F32),
                  jax.ShapeDtypeStruct((2, GACC, 16), F32)),
        mesh=mesh,
        compiler_params=pltpu.CompilerParams(use_tc_tiling_on_sc=False),
        scratch_types=[
            pltpu.VMEM((PGRP, 64), F32),
            pltpu.VMEM((1, PGRP), jnp.int32),
            pltpu.VMEM((PGRP, 16), F32),
            pltpu.VMEM((GR, 64), F32),
            pltpu.VMEM((GR, 16), F32),
            pltpu.VMEM_SHARED((GACC, 64), F32),
            pltpu.VMEM_SHARED((GACC, 16), F32),
        ],
    )
    return k(h3, batch2d)


def _full(shape):
    return pl.BlockSpec(shape, lambda i: (0,) * len(shape))


def _bmask(i):
    r = lax.broadcasted_iota(jnp.int32, (RB // 4, 128), 0)
    l = lax.broadcasted_iota(jnp.int32, (RB // 4, 128), 1)
    return (i * RB + 4 * r + l // 32) < N


def _fold4(q):
    return (q[:, 0:32] + q[:, 32:64] + q[:, 64:96] + q[:, 96:128])


def _dot(a, b):
    return jnp.dot(a, b, preferred_element_type=F32)


def _tc_l1a(xb, p1b, eps, K0, K1, ba0t, ba1t):
    def body(x_ref, p_ref, e_ref, k0_ref, k1_ref, b0_ref, b1_ref,
             u0_ref, u1_ref):
        z = (1.0 + e_ref[0, 0]) * x_ref[...] + p_ref[0] + p_ref[1]
        u0_ref[...] = jnp.maximum(_dot(z, k0_ref[...]) + b0_ref[...], 0.0)
        u1_ref[...] = jnp.maximum(_dot(z, k1_ref[...]) + b1_ref[...], 0.0)

    return pl.pallas_call(
        body, grid=(GRIDN,),
        in_specs=[pl.BlockSpec((RB // 8, 128), lambda i: (i, 0)),
                  pl.BlockSpec((2, RB // 8, 128), lambda i: (0, i, 0)),
                  _full((1, 1)), _full((128, 256)), _full((128, 256)),
                  _full((1, 256)), _full((1, 256))],
        out_specs=[pl.BlockSpec((RB // 8, 256), lambda i: (i, 0)),
                   pl.BlockSpec((RB // 8, 256), lambda i: (i, 0))],
        out_shape=[jax.ShapeDtypeStruct((NPAD // 8, 256), F32),
                   jax.ShapeDtypeStruct((NPAD // 8, 256), F32)],
    )(xb, p1b, eps, K0, K1, ba0t, ba1t)


def _tc_mlp(y0, y1, pA, pB, eps, KA, baT, KB, bbT, *, split):
    two_stage = pA is not None

    def body(*refs):
        i = pl.program_id(0)
        if two_stage:
            (y0_ref, y1_ref, pa_ref, pb_ref, e_ref, ka00, ka10, ka01, ka11,
             ba0, ba1, kb00, kb10, kb01, kb11, bb0, bb1, *outs) = refs
            s = 1.0 + e_ref[0, 0]
            z0 = s * y0_ref[...] + pa_ref[0] + pa_ref[1]
            z1 = s * y1_ref[...] + pb_ref[0] + pb_ref[1]
            u0 = jnp.maximum(_dot(z0, ka00[...]) + _dot(z1, ka10[...])
                             + ba0[...], 0.0)
            u1 = jnp.maximum(_dot(z0, ka01[...]) + _dot(z1, ka11[...])
                             + ba1[...], 0.0)
        else:
            (y0_ref, y1_ref, kb00, kb10, kb01, kb11, bb0, bb1, *outs) = refs
            u0 = y0_ref[...]
            u1 = y1_ref[...]
        v0 = jnp.maximum(_dot(u0, kb00[...]) + _dot(u1, kb10[...])
                         + bb0[...], 0.0)
        v1 = jnp.maximum(_dot(u0, kb01[...]) + _dot(u1, kb11[...])
                         + bb1[...], 0.0)
        m = _bmask(i)
        v0 = jnp.where(m, v0, 0.0)
        v1 = jnp.where(m, v1, 0.0)
        if split:
            h0_ref, h1_ref, ssum_ref, ssq_ref = outs

            @pl.when(i == 0)
            def _():
                ssum_ref[...] = jnp.zeros_like(ssum_ref)
                ssq_ref[...] = jnp.zeros_like(ssq_ref)
            h0_ref[...] = v0
            h1_ref[...] = v1
            ssum_ref[...] += jnp.concatenate(
                [_fold4(jnp.sum(v0, 0, keepdims=True)),
                 _fold4(jnp.sum(v1, 0, keepdims=True))], axis=1)
            ssq_ref[...] += jnp.concatenate(
                [_fold4(jnp.sum(v0 * v0, 0, keepdims=True)),
                 _fold4(jnp.sum(v1 * v1, 0, keepdims=True))], axis=1)
        else:
            outs[0][...] = jnp.concatenate(
                [v0[:, 0:32], v1[:, 0:32], v0[:, 32:64], v1[:, 32:64],
                 v0[:, 64:96], v1[:, 64:96], v0[:, 96:128], v1[:, 96:128]],
                axis=1)

    b32 = pl.BlockSpec((RB // 4, 128), lambda i: (i, 0))
    in_specs = [b32, b32]
    args = [y0, y1]
    if two_stage:
        in_specs += [pl.BlockSpec((2, RB // 4, 128), lambda i: (0, i, 0))] * 2
        in_specs += [_full((1, 1))] + [_full((128, 128))] * 4 \
            + [_full((1, 128))] * 2
        args += [pA.reshape(2, NPAD // 4, 128), pB.reshape(2, NPAD // 4, 128),
                 eps, *KA, *baT]
    in_specs += [_full((128, 128))] * 4 + [_full((1, 128))] * 2
    args += [*KB, *bbT]
    if split:
        out_specs = [b32, b32, _full((1, 64)), _full((1, 64))]
        out_shape = [jax.ShapeDtypeStruct((NPAD // 4, 128), F32),
                     jax.ShapeDtypeStruct((NPAD // 4, 128), F32),
                     jax.ShapeDtypeStruct((1, 64), F32),
                     jax.ShapeDtypeStruct((1, 64), F32)]
    else:
        out_specs = [pl.BlockSpec((RB // 4, 256), lambda i: (i, 0))]
        out_shape = [jax.ShapeDtypeStruct((NPAD // 4, 256), F32)]

    res = pl.pallas_call(body, grid=(GRIDN,), in_specs=in_specs,
                         out_specs=out_specs, out_shape=out_shape)(*args)
    return res if split else res[0]


def _tc_bn(h0, h1, ssum, ssq, g, be):
    def body(h0_ref, h1_ref, ssum_ref, ssq_ref, g_ref, be_ref, y0_ref, y1_ref):
        m = ssum_ref[...] / N
        v = ssq_ref[...] / N - m * m
        a = g_ref[...] * lax.rsqrt(v + 1e-5)
        sh = be_ref[...] - m * a
        a0 = jnp.concatenate([a[:, :32]] * 4, axis=1)
        a1 = jnp.concatenate([a[:, 32:]] * 4, axis=1)
        s0 = jnp.concatenate([sh[:, :32]] * 4, axis=1)
        s1 = jnp.concatenate([sh[:, 32:]] * 4, axis=1)
        y0_ref[...] = h0_ref[...] * a0 + s0
        y1_ref[...] = h1_ref[...] * a1 + s1

    return pl.pallas_call(
        body, grid=(GRIDN,),
        in_specs=[pl.BlockSpec((RB // 4, 128), lambda i: (i, 0)),
                  pl.BlockSpec((RB // 4, 128), lambda i: (i, 0)),
                  _full((1, 64)), _full((1, 64)), _full((1, 64)),
                  _full((1, 64))],
        out_specs=[pl.BlockSpec((RB // 4, 128), lambda i: (i, 0)),
                   pl.BlockSpec((RB // 4, 128), lambda i: (i, 0))],
        out_shape=[jax.ShapeDtypeStruct((NPAD // 4, 128), F32),
                   jax.ShapeDtypeStruct((NPAD // 4, 128), F32)],
    )(h0, h1, ssum, ssq, g, be)


def _tc_head(sums, cnts, g, be, w1, b1, w2, b2):
    def body(s_ref, c_ref, g_ref, be_ref, w1_ref, b1_ref, w2_ref, b2_ref,
             out_ref):
        s = (s_ref[0] + s_ref[1])[:1024]
        cnt = (c_ref[0] + c_ref[1])[:1024, 0:1]
        pooled = s / jnp.maximum(cnt, 1.0)
        m = jnp.mean(pooled, 0, keepdims=True)
        v = jnp.mean(pooled * pooled, 0, keepdims=True) - m * m
        pooled = (pooled - m) * lax.rsqrt(v + 1e-5) * g_ref[...] + be_ref[...]
        t = jnp.dot(pooled, w1_ref[...], preferred_element_type=F32) \
            + b1_ref[...]
        t = jnp.where(t > 0, t, jnp.exp(jnp.minimum(t, 0.0)) - 1.0)
        lo = jnp.dot(t, w2_ref[...], preferred_element_type=F32) + b2_ref[...]
        mx = jnp.max(lo, 1, keepdims=True)
        out_ref[...] = lo - mx - jnp.log(
            jnp.sum(jnp.exp(lo - mx), 1, keepdims=True))

    return pl.pallas_call(
        body,
        out_shape=jax.ShapeDtypeStruct((1024, 10), F32),
    )(sums, cnts, g, be, w1, b1, w2, b2)


def kernel(x, edge_index, batch, eps1, W1a, b1a, W1b, b1b, g1, be1,
           eps2, W2a, b2a, W2b, b2b, g2, be2,
           eps3, W3a, b3a, W3b, b3b, g3, be3,
           fc1W, fc1b, fc2W, fc2b):
    src = edge_index[0].astype(jnp.int32)
    dst = edge_index[1].astype(jnp.int32)
    pad = jnp.arange(EPAD - E, dtype=jnp.int32) % 1024
    src2d = jnp.concatenate([src, pad]).reshape(NGRP, GRP)
    dst2d = jnp.concatenate([dst, N + pad]).reshape(NGRP, GRP)
    xb = jnp.pad(x, ((0, NPAD - N), (0, 13))).reshape(NPAD // 8, 128)
    b2d = jnp.concatenate(
        [batch.astype(jnp.int32),
         jnp.full((NPAD - N,), 1024, jnp.int32)]).reshape(PG, PGRP)
    r1 = lambda a: a.reshape(1, -1)
    e1, e2, e3 = (jnp.reshape(e, (1, 1)) for e in (eps1, eps2, eps3))
    W1ap = jnp.pad(W1a, ((0, 13), (0, 0)))
    i4 = jnp.eye(4, dtype=F32)
    i8 = jnp.eye(8, dtype=F32)

    def kq(W):
        return tuple(jnp.kron(i4, W[a:a + 32, b:b + 32])
                     for b in (0, 32) for a in (0, 32))

    def tb4(b):
        return (jnp.tile(b[:32].reshape(1, 32), (1, 4)),
                jnp.tile(b[32:].reshape(1, 32), (1, 4)))

    p1 = _sc_agg(xb.reshape(NPAD, 16), src2d, dst2d, 16)
    u0, u1 = _tc_l1a(xb, p1.reshape(2, NPAD // 8, 128), e1,
                     jnp.kron(i8, W1ap[:, :32]), jnp.kron(i8, W1ap[:, 32:]),
                     jnp.tile(b1a[:32].reshape(1, 32), (1, 8)),
                     jnp.tile(b1a[32:].reshape(1, 32), (1, 8)))
    h0, h1, ss, sq = _tc_mlp(u0.reshape(NPAD // 4, 128),
                             u1.reshape(NPAD // 4, 128),
                             None, None, None, None, None,
                             kq(W1b), tb4(b1b), split=True)
    y0, y1 = _tc_bn(h0, h1, ss, sq, r1(g1), r1(be1))

    pA = _sc_agg(y0.reshape(NPAD, 32), src2d, dst2d, 32)
    pB = _sc_agg(y1.reshape(NPAD, 32), src2d, dst2d, 32)
    h0, h1, ss, sq = _tc_mlp(y0, y1, pA, pB, e2, kq(W2a), tb4(b2a),
                             kq(W2b), tb4(b2b), split=True)
    y0, y1 = _tc_bn(h0, h1, ss, sq, r1(g2), r1(be2))

    pA = _sc_agg(y0.reshape(NPAD, 32), src2d, dst2d, 32)
    pB = _sc_agg(y1.reshape(NPAD, 32), src2d, dst2d, 32)
    h3 = _tc_mlp(y0, y1, pA, pB, e3, kq(W3a), tb4(b3a),
                 kq(W3b), tb4(b3b), split=False)

    sums, cnts = _sc_pool(h3.reshape(NPAD, 64), b2d)
    return _tc_head(sums, cnts, r1(g3), r1(be3), fc1W, r1(fc1b), fc2W,
                    r1(fc2b))


# speedup vs baseline: 1.2621x; 1.2621x over previous
import jax
import jax.numpy as jnp
from jax import lax
from jax.experimental import pallas as pl
from jax.experimental.pallas import tpu as pltpu
from jax.experimental.pallas import tpu_sc as plsc

N = 50000
NPAD = 53248
E = 800000
EPAD = 819200
GRP = 128
NGRP = EPAD // GRP
NW = 32
GPW = NGRP // NW
IC = 20
NBLK = GPW // IC
NCH = IC // 2
ROWS_PT = NPAD // 16
NZDMA = 8
ZR = 416
GACC = 1056
PGRP = 128
PG = NPAD // PGRP
PGPW = PG // NW
RB = 1024
GRIDN = NPAD // RB
F32 = jnp.float32


def _sc_agg(table, src2d, dst2d, F):
    mesh = plsc.VectorSubcoreMesh(core_axis_name="c", subcore_axis_name="s")
    fl = F // 16

    def body(table_ref, src_ref, dst_ref, out_ref, sidx, didx, rows,
             acc, sg0, sg1, ss0, ss1):
        c = lax.axis_index("c")
        s = lax.axis_index("s")
        w = c * 16 + s
        zv = jnp.zeros((16,), F32)
        semg = (sg0, sg1)
        sems = (ss0, ss1)

        def zinit(i, _):
            rows[i // fl, pl.ds((i % fl) * 16, 16)] = zv
            return 0
        lax.fori_loop(0, 2 * GRP * fl, zinit, 0)

        def zdma(q, _):
            pltpu.sync_copy(rows.at[pl.ds(0, ZR // 2)],
                            acc.at[pl.ds(s * ROWS_PT + q * (ZR // 2),
                                         ZR // 2)])
            return 0
        lax.fori_loop(0, 2 * NZDMA, zdma, 0)
        plsc.subcore_barrier()

        def gath(pc, j):
            par = pc % 2
            return pltpu.async_copy(
                table_ref.at[sidx.at[2 * pc + j]],
                rows.at[pl.ds((par * 2 + j) * GRP, GRP)], semg[par])

        def scat(pc, j):
            par = pc % 2
            return pltpu.async_copy(
                rows.at[pl.ds((par * 2 + j) * GRP, GRP)],
                acc.at[didx.at[2 * pc + j]], sems[par], add=True)

        def block(ib, _):
            g0 = w * GPW + ib * IC
            pltpu.sync_copy(src_ref.at[pl.ds(g0, IC)], sidx)
            pltpu.sync_copy(dst_ref.at[pl.ds(g0, IC)], didx)
            gd = {0: [gath(0, 0), gath(0, 1)]}
            sd = {}
            for pc in range(NCH):
                for d in gd.pop(pc):
                    d.wait()
                if pc >= 1:
                    for d in sd.pop(pc - 1):
                        d.wait()
                if pc < NCH - 1:
                    gd[pc + 1] = [gath(pc + 1, 0), gath(pc + 1, 1)]
                sd[pc] = [scat(pc, 0), scat(pc, 1)]
            for d in sd.pop(NCH - 1):
                d.wait()
            return 0
        lax.fori_loop(0, NBLK, block, 0)
        plsc.subcore_barrier()
        pltpu.sync_copy(acc.at[pl.ds(s * ROWS_PT, ROWS_PT)],
                        out_ref.at[c, pl.ds(s * ROWS_PT, ROWS_PT)])

    k = pl.kernel(
        body,
        out_type=jax.ShapeDtypeStruct((2, NPAD, F), F32),
        mesh=mesh,
        compiler_params=pltpu.CompilerParams(use_tc_tiling_on_sc=False),
        scratch_types=[
            pltpu.VMEM((IC, GRP), jnp.int32),
            pltpu.VMEM((IC, GRP), jnp.int32),
            pltpu.VMEM((4 * GRP, F), F32),
            pltpu.VMEM_SHARED((NPAD, F), F32),
            pltpu.SemaphoreType.DMA,
            pltpu.SemaphoreType.DMA,
            pltpu.SemaphoreType.DMA,
            pltpu.SemaphoreType.DMA,
        ],
    )
    return k(table, src2d, dst2d)


def _sc_pool(h3, batch2d):
    mesh = plsc.VectorSubcoreMesh(core_axis_name="c", subcore_axis_name="s")
    GR = GACC // 16

    def body(h_ref, b_ref, outs_ref, outc_ref, rows, bidx, ones_v, zbs, zbc,
             accs, accc):
        c = lax.axis_index("c")
        s = lax.axis_index("s")
        w = c * 16 + s
        zv = jnp.zeros((16,), F32)
        ov = jnp.full((16,), 1.0, F32)

        def oinit(i, _):
            ones_v[i, pl.ds(0, 16)] = ov
            return 0
        lax.fori_loop(0, PGRP, oinit, 0)

        def zsinit(i, _):
            zbs[i // 4, pl.ds((i % 4) * 16, 16)] = zv
            return 0
        lax.fori_loop(0, GR * 4, zsinit, 0)

        def zcinit(i, _):
            zbc[i, pl.ds(0, 16)] = zv
            return 0
        lax.fori_loop(0, GR, zcinit, 0)
        pltpu.sync_copy(zbs, accs.at[pl.ds(s * GR, GR)])
        pltpu.sync_copy(zbc, accc.at[pl.ds(s * GR, GR)])
        plsc.subcore_barrier()

        def grp(k, _):
            g = w * PGPW + k
            pltpu.sync_copy(b_ref.at[pl.ds(g, 1)], bidx)
            pltpu.sync_copy(h_ref.at[pl.ds(g * PGRP, PGRP)], rows)
            pltpu.sync_copy(rows, accs.at[bidx.at[0]], add=True)
            pltpu.sync_copy(ones_v, accc.at[bidx.at[0]], add=True)
            return 0
        lax.fori_loop(0, PGPW, grp, 0)
        plsc.subcore_barrier()

        @pl.when(s == 0)
        def _():
            pltpu.sync_copy(accs, outs_ref.at[c])
            pltpu.sync_copy(accc, outc_ref.at[c])

    k = pl.kernel(
        body,
        out_type=(jax.ShapeDtypeStruct((2, GACC, 64), F32),
                  jax.ShapeDtypeStruct((2, GACC, 16), F32)),
        mesh=mesh,
        compiler_params=pltpu.CompilerParams(use_tc_tiling_on_sc=False),
        scratch_types=[
            pltpu.VMEM((PGRP, 64), F32),
            pltpu.VMEM((1, PGRP), jnp.int32),
            pltpu.VMEM((PGRP, 16), F32),
            pltpu.VMEM((GR, 64), F32),
            pltpu.VMEM((GR, 16), F32),
            pltpu.VMEM_SHARED((GACC, 64), F32),
            pltpu.VMEM_SHARED((GACC, 16), F32),
        ],
    )
    return k(h3, batch2d)


def _full(shape):
    return pl.BlockSpec(shape, lambda i: (0,) * len(shape))


def _bmask(i):
    r = lax.broadcasted_iota(jnp.int32, (RB // 4, 128), 0)
    l = lax.broadcasted_iota(jnp.int32, (RB // 4, 128), 1)
    return (i * RB + 4 * r + l // 32) < N


def _fold4(q):
    return (q[:, 0:32] + q[:, 32:64] + q[:, 64:96] + q[:, 96:128])


def _dot(a, b):
    return jnp.dot(a, b, preferred_element_type=F32)


def _tc_l1a(xb, p1b, eps, K0, K1, ba0t, ba1t):
    def body(x_ref, p_ref, e_ref, k0_ref, k1_ref, b0_ref, b1_ref,
             u0_ref, u1_ref):
        z = (1.0 + e_ref[0, 0]) * x_ref[...] + p_ref[0] + p_ref[1]
        u0_ref[...] = jnp.maximum(_dot(z, k0_ref[...]) + b0_ref[...], 0.0)
        u1_ref[...] = jnp.maximum(_dot(z, k1_ref[...]) + b1_ref[...], 0.0)

    return pl.pallas_call(
        body, grid=(GRIDN,),
        in_specs=[pl.BlockSpec((RB // 8, 128), lambda i: (i, 0)),
                  pl.BlockSpec((2, RB // 8, 128), lambda i: (0, i, 0)),
                  _full((1, 1)), _full((128, 256)), _full((128, 256)),
                  _full((1, 256)), _full((1, 256))],
        out_specs=[pl.BlockSpec((RB // 8, 256), lambda i: (i, 0)),
                   pl.BlockSpec((RB // 8, 256), lambda i: (i, 0))],
        out_shape=[jax.ShapeDtypeStruct((NPAD // 8, 256), F32),
                   jax.ShapeDtypeStruct((NPAD // 8, 256), F32)],
    )(xb, p1b, eps, K0, K1, ba0t, ba1t)


def _tc_mlp(y0, y1, pA, pB, eps, KA, baT, KB, bbT, *, split):
    two_stage = pA is not None

    def body(*refs):
        i = pl.program_id(0)
        if two_stage:
            (y0_ref, y1_ref, pa_ref, pb_ref, e_ref, ka00, ka10, ka01, ka11,
             ba0, ba1, kb00, kb10, kb01, kb11, bb0, bb1, *outs) = refs
            s = 1.0 + e_ref[0, 0]
            z0 = s * y0_ref[...] + pa_ref[0] + pa_ref[1]
            z1 = s * y1_ref[...] + pb_ref[0] + pb_ref[1]
            u0 = jnp.maximum(_dot(z0, ka00[...]) + _dot(z1, ka10[...])
                             + ba0[...], 0.0)
            u1 = jnp.maximum(_dot(z0, ka01[...]) + _dot(z1, ka11[...])
                             + ba1[...], 0.0)
        else:
            (y0_ref, y1_ref, kb00, kb10, kb01, kb11, bb0, bb1, *outs) = refs
            u0 = y0_ref[...]
            u1 = y1_ref[...]
        v0 = jnp.maximum(_dot(u0, kb00[...]) + _dot(u1, kb10[...])
                         + bb0[...], 0.0)
        v1 = jnp.maximum(_dot(u0, kb01[...]) + _dot(u1, kb11[...])
                         + bb1[...], 0.0)
        m = _bmask(i)
        v0 = jnp.where(m, v0, 0.0)
        v1 = jnp.where(m, v1, 0.0)
        if split:
            h0_ref, h1_ref, ssum_ref, ssq_ref = outs

            @pl.when(i == 0)
            def _():
                ssum_ref[...] = jnp.zeros_like(ssum_ref)
                ssq_ref[...] = jnp.zeros_like(ssq_ref)
            h0_ref[...] = v0
            h1_ref[...] = v1
            ssum_ref[...] += jnp.concatenate(
                [_fold4(jnp.sum(v0, 0, keepdims=True)),
                 _fold4(jnp.sum(v1, 0, keepdims=True))], axis=1)
            ssq_ref[...] += jnp.concatenate(
                [_fold4(jnp.sum(v0 * v0, 0, keepdims=True)),
                 _fold4(jnp.sum(v1 * v1, 0, keepdims=True))], axis=1)
        else:
            outs[0][...] = jnp.concatenate(
                [v0[:, 0:32], v1[:, 0:32], v0[:, 32:64], v1[:, 32:64],
                 v0[:, 64:96], v1[:, 64:96], v0[:, 96:128], v1[:, 96:128]],
                axis=1)

    b32 = pl.BlockSpec((RB // 4, 128), lambda i: (i, 0))
    in_specs = [b32, b32]
    args = [y0, y1]
    if two_stage:
        in_specs += [pl.BlockSpec((2, RB // 4, 128), lambda i: (0, i, 0))] * 2
        in_specs += [_full((1, 1))] + [_full((128, 128))] * 4 \
            + [_full((1, 128))] * 2
        args += [pA.reshape(2, NPAD // 4, 128), pB.reshape(2, NPAD // 4, 128),
                 eps, *KA, *baT]
    in_specs += [_full((128, 128))] * 4 + [_full((1, 128))] * 2
    args += [*KB, *bbT]
    if split:
        out_specs = [b32, b32, _full((1, 64)), _full((1, 64))]
        out_shape = [jax.ShapeDtypeStruct((NPAD // 4, 128), F32),
                     jax.ShapeDtypeStruct((NPAD // 4, 128), F32),
                     jax.ShapeDtypeStruct((1, 64), F32),
                     jax.ShapeDtypeStruct((1, 64), F32)]
    else:
        out_specs = [pl.BlockSpec((RB // 4, 256), lambda i: (i, 0))]
        out_shape = [jax.ShapeDtypeStruct((NPAD // 4, 256), F32)]

    res = pl.pallas_call(body, grid=(GRIDN,), in_specs=in_specs,
                         out_specs=out_specs, out_shape=out_shape)(*args)
    return res if split else res[0]


def _tc_bn(h0, h1, ssum, ssq, g, be):
    def body(h0_ref, h1_ref, ssum_ref, ssq_ref, g_ref, be_ref, y0_ref, y1_ref):
        m = ssum_ref[...] / N
        v = ssq_ref[...] / N - m * m
        a = g_ref[...] * lax.rsqrt(v + 1e-5)
        sh = be_ref[...] - m * a
        a0 = jnp.concatenate([a[:, :32]] * 4, axis=1)
        a1 = jnp.concatenate([a[:, 32:]] * 4, axis=1)
        s0 = jnp.concatenate([sh[:, :32]] * 4, axis=1)
        s1 = jnp.concatenate([sh[:, 32:]] * 4, axis=1)
        y0_ref[...] = h0_ref[...] * a0 + s0
        y1_ref[...] = h1_ref[...] * a1 + s1

    return pl.pallas_call(
        body, grid=(GRIDN,),
        in_specs=[pl.BlockSpec((RB // 4, 128), lambda i: (i, 0)),
                  pl.BlockSpec((RB // 4, 128), lambda i: (i, 0)),
                  _full((1, 64)), _full((1, 64)), _full((1, 64)),
                  _full((1, 64))],
        out_specs=[pl.BlockSpec((RB // 4, 128), lambda i: (i, 0)),
                   pl.BlockSpec((RB // 4, 128), lambda i: (i, 0))],
        out_shape=[jax.ShapeDtypeStruct((NPAD // 4, 128), F32),
                   jax.ShapeDtypeStruct((NPAD // 4, 128), F32)],
    )(h0, h1, ssum, ssq, g, be)


def _tc_head(sums, cnts, g, be, w1, b1, w2, b2):
    def body(s_ref, c_ref, g_ref, be_ref, w1_ref, b1_ref, w2_ref, b2_ref,
             out_ref):
        s = (s_ref[0] + s_ref[1])[:1024]
        cnt = (c_ref[0] + c_ref[1])[:1024, 0:1]
        pooled = s / jnp.maximum(cnt, 1.0)
        m = jnp.mean(pooled, 0, keepdims=True)
        v = jnp.mean(pooled * pooled, 0, keepdims=True) - m * m
        pooled = (pooled - m) * lax.rsqrt(v + 1e-5) * g_ref[...] + be_ref[...]
        t = jnp.dot(pooled, w1_ref[...], preferred_element_type=F32) \
            + b1_ref[...]
        t = jnp.where(t > 0, t, jnp.exp(jnp.minimum(t, 0.0)) - 1.0)
        lo = jnp.dot(t, w2_ref[...], preferred_element_type=F32) + b2_ref[...]
        mx = jnp.max(lo, 1, keepdims=True)
        out_ref[...] = lo - mx - jnp.log(
            jnp.sum(jnp.exp(lo - mx), 1, keepdims=True))

    return pl.pallas_call(
        body,
        out_shape=jax.ShapeDtypeStruct((1024, 10), F32),
    )(sums, cnts, g, be, w1, b1, w2, b2)


def kernel(x, edge_index, batch, eps1, W1a, b1a, W1b, b1b, g1, be1,
           eps2, W2a, b2a, W2b, b2b, g2, be2,
           eps3, W3a, b3a, W3b, b3b, g3, be3,
           fc1W, fc1b, fc2W, fc2b):
    src = edge_index[0].astype(jnp.int32)
    dst = edge_index[1].astype(jnp.int32)
    pad = jnp.arange(EPAD - E, dtype=jnp.int32) % 1024
    src2d = jnp.concatenate([src, pad]).reshape(NGRP, GRP)
    dst2d = jnp.concatenate([dst, N + pad]).reshape(NGRP, GRP)
    xb = jnp.pad(x, ((0, NPAD - N), (0, 13))).reshape(NPAD // 8, 128)
    b2d = jnp.concatenate(
        [batch.astype(jnp.int32),
         jnp.full((NPAD - N,), 1024, jnp.int32)]).reshape(PG, PGRP)
    r1 = lambda a: a.reshape(1, -1)
    e1, e2, e3 = (jnp.reshape(e, (1, 1)) for e in (eps1, eps2, eps3))
    W1ap = jnp.pad(W1a, ((0, 13), (0, 0)))
    i4 = jnp.eye(4, dtype=F32)
    i8 = jnp.eye(8, dtype=F32)

    def kq(W):
        return tuple(jnp.kron(i4, W[a:a + 32, b:b + 32])
                     for b in (0, 32) for a in (0, 32))

    def tb4(b):
        return (jnp.tile(b[:32].reshape(1, 32), (1, 4)),
                jnp.tile(b[32:].reshape(1, 32), (1, 4)))

    p1 = _sc_agg(xb.reshape(NPAD, 16), src2d, dst2d, 16)
    u0, u1 = _tc_l1a(xb, p1.reshape(2, NPAD // 8, 128), e1,
                     jnp.kron(i8, W1ap[:, :32]), jnp.kron(i8, W1ap[:, 32:]),
                     jnp.tile(b1a[:32].reshape(1, 32), (1, 8)),
                     jnp.tile(b1a[32:].reshape(1, 32), (1, 8)))
    h0, h1, ss, sq = _tc_mlp(u0.reshape(NPAD // 4, 128),
                             u1.reshape(NPAD // 4, 128),
                             None, None, None, None, None,
                             kq(W1b), tb4(b1b), split=True)
    y0, y1 = _tc_bn(h0, h1, ss, sq, r1(g1), r1(be1))

    pA = _sc_agg(y0.reshape(NPAD, 32), src2d, dst2d, 32)
    pB = _sc_agg(y1.reshape(NPAD, 32), src2d, dst2d, 32)
    h0, h1, ss, sq = _tc_mlp(y0, y1, pA, pB, e2, kq(W2a), tb4(b2a),
                             kq(W2b), tb4(b2b), split=True)
    y0, y1 = _tc_bn(h0, h1, ss, sq, r1(g2), r1(be2))

    pA = _sc_agg(y0.reshape(NPAD, 32), src2d, dst2d, 32)
    pB = _sc_agg(y1.reshape(NPAD, 32), src2d, dst2d, 32)
    h3 = _tc_mlp(y0, y1, pA, pB, e3, kq(W3a), tb4(b3a),
                 kq(W3b), tb4(b3b), split=False)

    sums, cnts = _sc_pool(h3.reshape(NPAD, 64), b2d)
    return _tc_head(sums, cnts, r1(g3), r1(be3), fc1W, r1(fc1b), fc2W,
                    r1(fc2b))


# speedup vs baseline: 1.3362x; 1.0588x over previous
import jax
import jax.numpy as jnp
from jax import lax
from jax.experimental import pallas as pl
from jax.experimental.pallas import tpu as pltpu
from jax.experimental.pallas import tpu_sc as plsc

N = 50000
NPAD = 53248
E = 800000
EPAD = 819200
GRP = 128
NGRP = EPAD // GRP
NW = 32
GPW = NGRP // NW
IC = 40
NBLK = GPW // IC
NACC = 51200
ROWS_PT = NACC // 16
NZDMA = 8
ZR = 400
GACC = 1056
PGRP = 128
PG = NPAD // PGRP
PGPW = PG // NW
RB = 1024
GRIDN = NPAD // RB
F32 = jnp.float32


def _sc_agg(table, src2d, dst2d, F):
    mesh = plsc.VectorSubcoreMesh(core_axis_name="c", subcore_axis_name="s")
    fl = F // 16
    cw = 4 if F == 16 else 2
    nch = IC // cw

    def body(table_ref, src_ref, dst_ref, out_ref, sidx, didx, rows,
             acc, sg0, sg1, ss0, ss1):
        c = lax.axis_index("c")
        s = lax.axis_index("s")
        w = c * 16 + s
        zv = jnp.zeros((16,), F32)
        semg = (sg0, sg1)
        sems = (ss0, ss1)

        def zinit(i, _):
            rows[i // fl, pl.ds((i % fl) * 16, 16)] = zv
            return 0
        lax.fori_loop(0, 2 * GRP * fl, zinit, 0)

        def zdma(q, _):
            pltpu.sync_copy(rows.at[pl.ds(0, ZR // 2)],
                            acc.at[pl.ds(s * ROWS_PT + q * (ZR // 2),
                                         ZR // 2)])
            return 0
        lax.fori_loop(0, 2 * NZDMA, zdma, 0)
        plsc.subcore_barrier()

        def gath(pc, j):
            par = pc % 2
            return pltpu.async_copy(
                table_ref.at[sidx.at[cw * pc + j]],
                rows.at[pl.ds((par * cw + j) * GRP, GRP)], semg[par])

        def scat(pc, j):
            par = pc % 2
            return pltpu.async_copy(
                rows.at[pl.ds((par * cw + j) * GRP, GRP)],
                acc.at[didx.at[cw * pc + j]], sems[par], add=True)

        def block(ib, _):
            g0 = w * GPW + ib * IC
            pltpu.sync_copy(src_ref.at[pl.ds(g0, IC)], sidx)
            pltpu.sync_copy(dst_ref.at[pl.ds(g0, IC)], didx)
            gd = {0: [gath(0, j) for j in range(cw)]}
            sd = {}
            for pc in range(nch):
                for d in gd.pop(pc):
                    d.wait()
                if pc >= 1:
                    for d in sd.pop(pc - 1):
                        d.wait()
                if pc < nch - 1:
                    gd[pc + 1] = [gath(pc + 1, j) for j in range(cw)]
                sd[pc] = [scat(pc, j) for j in range(cw)]
            for d in sd.pop(nch - 1):
                d.wait()
            return 0
        lax.fori_loop(0, NBLK, block, 0)
        plsc.subcore_barrier()
        pltpu.sync_copy(acc.at[pl.ds(s * ROWS_PT, ROWS_PT)],
                        out_ref.at[c, pl.ds(s * ROWS_PT, ROWS_PT)])

    k = pl.kernel(
        body,
        out_type=jax.ShapeDtypeStruct((2, NPAD, F), F32),
        mesh=mesh,
        compiler_params=pltpu.CompilerParams(use_tc_tiling_on_sc=False),
        scratch_types=[
            pltpu.VMEM((IC, GRP), jnp.int32),
            pltpu.VMEM((IC, GRP), jnp.int32),
            pltpu.VMEM((2 * cw * GRP, F), F32),
            pltpu.VMEM_SHARED((NACC, F), F32),
            pltpu.SemaphoreType.DMA,
            pltpu.SemaphoreType.DMA,
            pltpu.SemaphoreType.DMA,
            pltpu.SemaphoreType.DMA,
        ],
    )
    return k(table, src2d, dst2d)


def _sc_pool(h3, batch2d):
    mesh = plsc.VectorSubcoreMesh(core_axis_name="c", subcore_axis_name="s")
    GR = GACC // 16

    def body(h_ref, b_ref, outs_ref, outc_ref, rows, bidx, ones_v, zbs, zbc,
             accs, accc):
        c = lax.axis_index("c")
        s = lax.axis_index("s")
        w = c * 16 + s
        zv = jnp.zeros((16,), F32)
        ov = jnp.full((16,), 1.0, F32)

        def oinit(i, _):
            ones_v[i, pl.ds(0, 16)] = ov
            return 0
        lax.fori_loop(0, PGRP, oinit, 0)

        def zsinit(i, _):
            zbs[i // 4, pl.ds((i % 4) * 16, 16)] = zv
            return 0
        lax.fori_loop(0, GR * 4, zsinit, 0)

        def zcinit(i, _):
            zbc[i, pl.ds(0, 16)] = zv
            return 0
        lax.fori_loop(0, GR, zcinit, 0)
        pltpu.sync_copy(zbs, accs.at[pl.ds(s * GR, GR)])
        pltpu.sync_copy(zbc, accc.at[pl.ds(s * GR, GR)])
        plsc.subcore_barrier()

        def grp(k, _):
            g = w * PGPW + k
            pltpu.sync_copy(b_ref.at[pl.ds(g, 1)], bidx)
            pltpu.sync_copy(h_ref.at[pl.ds(g * PGRP, PGRP)], rows)
            pltpu.sync_copy(rows, accs.at[bidx.at[0]], add=True)
            pltpu.sync_copy(ones_v, accc.at[bidx.at[0]], add=True)
            return 0
        lax.fori_loop(0, PGPW, grp, 0)
        plsc.subcore_barrier()

        @pl.when(s == 0)
        def _():
            pltpu.sync_copy(accs, outs_ref.at[c])
            pltpu.sync_copy(accc, outc_ref.at[c])

    k = pl.kernel(
        body,
        out_type=(jax.ShapeDtypeStruct((2, GACC, 64), F32),
                  jax.ShapeDtypeStruct((2, GACC, 16), F32)),
        mesh=mesh,
        compiler_params=pltpu.CompilerParams(use_tc_tiling_on_sc=False),
        scratch_types=[
            pltpu.VMEM((PGRP, 64), F32),
            pltpu.VMEM((1, PGRP), jnp.int32),
            pltpu.VMEM((PGRP, 16), F32),
            pltpu.VMEM((GR, 64), F32),
            pltpu.VMEM((GR, 16), F32),
            pltpu.VMEM_SHARED((GACC, 64), F32),
            pltpu.VMEM_SHARED((GACC, 16), F32),
        ],
    )
    return k(h3, batch2d)


def _full(shape):
    return pl.BlockSpec(shape, lambda i: (0,) * len(shape))


def _bmask(i):
    r = lax.broadcasted_iota(jnp.int32, (RB // 4, 128), 0)
    l = lax.broadcasted_iota(jnp.int32, (RB // 4, 128), 1)
    return (i * RB + 4 * r + l // 32) < N


def _fold4(q):
    return (q[:, 0:32] + q[:, 32:64] + q[:, 64:96] + q[:, 96:128])


def _dot(a, b):
    return jnp.dot(a, b, preferred_element_type=F32)


def _tc_l1a(xb, p1b, eps, K0, K1, ba0t, ba1t):
    def body(x_ref, p_ref, e_ref, k0_ref, k1_ref, b0_ref, b1_ref,
             u0_ref, u1_ref):
        z = (1.0 + e_ref[0, 0]) * x_ref[...] + p_ref[0] + p_ref[1]
        u0_ref[...] = jnp.maximum(_dot(z, k0_ref[...]) + b0_ref[...], 0.0)
        u1_ref[...] = jnp.maximum(_dot(z, k1_ref[...]) + b1_ref[...], 0.0)

    return pl.pallas_call(
        body, grid=(GRIDN,),
        in_specs=[pl.BlockSpec((RB // 8, 128), lambda i: (i, 0)),
                  pl.BlockSpec((2, RB // 8, 128), lambda i: (0, i, 0)),
                  _full((1, 1)), _full((128, 256)), _full((128, 256)),
                  _full((1, 256)), _full((1, 256))],
        out_specs=[pl.BlockSpec((RB // 8, 256), lambda i: (i, 0)),
                   pl.BlockSpec((RB // 8, 256), lambda i: (i, 0))],
        out_shape=[jax.ShapeDtypeStruct((NPAD // 8, 256), F32),
                   jax.ShapeDtypeStruct((NPAD // 8, 256), F32)],
    )(xb, p1b, eps, K0, K1, ba0t, ba1t)


def _tc_mlp(y0, y1, pA, pB, eps, KA, baT, KB, bbT, *, split):
    two_stage = pA is not None

    def body(*refs):
        i = pl.program_id(0)
        if two_stage:
            (y0_ref, y1_ref, pa_ref, pb_ref, e_ref, ka00, ka10, ka01, ka11,
             ba0, ba1, kb00, kb10, kb01, kb11, bb0, bb1, *outs) = refs
            s = 1.0 + e_ref[0, 0]
            z0 = s * y0_ref[...] + pa_ref[0] + pa_ref[1]
            z1 = s * y1_ref[...] + pb_ref[0] + pb_ref[1]
            u0 = jnp.maximum(_dot(z0, ka00[...]) + _dot(z1, ka10[...])
                             + ba0[...], 0.0)
            u1 = jnp.maximum(_dot(z0, ka01[...]) + _dot(z1, ka11[...])
                             + ba1[...], 0.0)
        else:
            (y0_ref, y1_ref, kb00, kb10, kb01, kb11, bb0, bb1, *outs) = refs
            u0 = y0_ref[...]
            u1 = y1_ref[...]
        v0 = jnp.maximum(_dot(u0, kb00[...]) + _dot(u1, kb10[...])
                         + bb0[...], 0.0)
        v1 = jnp.maximum(_dot(u0, kb01[...]) + _dot(u1, kb11[...])
                         + bb1[...], 0.0)
        m = _bmask(i)
        v0 = jnp.where(m, v0, 0.0)
        v1 = jnp.where(m, v1, 0.0)
        if split:
            h0_ref, h1_ref, ssum_ref, ssq_ref = outs

            @pl.when(i == 0)
            def _():
                ssum_ref[...] = jnp.zeros_like(ssum_ref)
                ssq_ref[...] = jnp.zeros_like(ssq_ref)
            h0_ref[...] = v0
            h1_ref[...] = v1
            ssum_ref[...] += jnp.concatenate(
                [_fold4(jnp.sum(v0, 0, keepdims=True)),
                 _fold4(jnp.sum(v1, 0, keepdims=True))], axis=1)
            ssq_ref[...] += jnp.concatenate(
                [_fold4(jnp.sum(v0 * v0, 0, keepdims=True)),
                 _fold4(jnp.sum(v1 * v1, 0, keepdims=True))], axis=1)
        else:
            outs[0][...] = jnp.concatenate(
                [v0[:, 0:32], v1[:, 0:32], v0[:, 32:64], v1[:, 32:64],
                 v0[:, 64:96], v1[:, 64:96], v0[:, 96:128], v1[:, 96:128]],
                axis=1)

    b32 = pl.BlockSpec((RB // 4, 128), lambda i: (i, 0))
    in_specs = [b32, b32]
    args = [y0, y1]
    if two_stage:
        in_specs += [pl.BlockSpec((2, RB // 4, 128), lambda i: (0, i, 0))] * 2
        in_specs += [_full((1, 1))] + [_full((128, 128))] * 4 \
            + [_full((1, 128))] * 2
        args += [pA.reshape(2, NPAD // 4, 128), pB.reshape(2, NPAD // 4, 128),
                 eps, *KA, *baT]
    in_specs += [_full((128, 128))] * 4 + [_full((1, 128))] * 2
    args += [*KB, *bbT]
    if split:
        out_specs = [b32, b32, _full((1, 64)), _full((1, 64))]
        out_shape = [jax.ShapeDtypeStruct((NPAD // 4, 128), F32),
                     jax.ShapeDtypeStruct((NPAD // 4, 128), F32),
                     jax.ShapeDtypeStruct((1, 64), F32),
                     jax.ShapeDtypeStruct((1, 64), F32)]
    else:
        out_specs = [pl.BlockSpec((RB // 4, 256), lambda i: (i, 0))]
        out_shape = [jax.ShapeDtypeStruct((NPAD // 4, 256), F32)]

    res = pl.pallas_call(body, grid=(GRIDN,), in_specs=in_specs,
                         out_specs=out_specs, out_shape=out_shape)(*args)
    return res if split else res[0]


def _tc_bn(h0, h1, ssum, ssq, g, be):
    def body(h0_ref, h1_ref, ssum_ref, ssq_ref, g_ref, be_ref, y0_ref, y1_ref):
        m = ssum_ref[...] / N
        v = ssq_ref[...] / N - m * m
        a = g_ref[...] * lax.rsqrt(v + 1e-5)
        sh = be_ref[...] - m * a
        a0 = jnp.concatenate([a[:, :32]] * 4, axis=1)
        a1 = jnp.concatenate([a[:, 32:]] * 4, axis=1)
        s0 = jnp.concatenate([sh[:, :32]] * 4, axis=1)
        s1 = jnp.concatenate([sh[:, 32:]] * 4, axis=1)
        y0_ref[...] = h0_ref[...] * a0 + s0
        y1_ref[...] = h1_ref[...] * a1 + s1

    return pl.pallas_call(
        body, grid=(GRIDN,),
        in_specs=[pl.BlockSpec((RB // 4, 128), lambda i: (i, 0)),
                  pl.BlockSpec((RB // 4, 128), lambda i: (i, 0)),
                  _full((1, 64)), _full((1, 64)), _full((1, 64)),
                  _full((1, 64))],
        out_specs=[pl.BlockSpec((RB // 4, 128), lambda i: (i, 0)),
                   pl.BlockSpec((RB // 4, 128), lambda i: (i, 0))],
        out_shape=[jax.ShapeDtypeStruct((NPAD // 4, 128), F32),
                   jax.ShapeDtypeStruct((NPAD // 4, 128), F32)],
    )(h0, h1, ssum, ssq, g, be)


def _tc_head(sums, cnts, g, be, w1, b1, w2, b2):
    def body(s_ref, c_ref, g_ref, be_ref, w1_ref, b1_ref, w2_ref, b2_ref,
             out_ref):
        s = (s_ref[0] + s_ref[1])[:1024]
        cnt = (c_ref[0] + c_ref[1])[:1024, 0:1]
        pooled = s / jnp.maximum(cnt, 1.0)
        m = jnp.mean(pooled, 0, keepdims=True)
        v = jnp.mean(pooled * pooled, 0, keepdims=True) - m * m
        pooled = (pooled - m) * lax.rsqrt(v + 1e-5) * g_ref[...] + be_ref[...]
        t = jnp.dot(pooled, w1_ref[...], preferred_element_type=F32) \
            + b1_ref[...]
        t = jnp.where(t > 0, t, jnp.exp(jnp.minimum(t, 0.0)) - 1.0)
        lo = jnp.dot(t, w2_ref[...], preferred_element_type=F32) + b2_ref[...]
        mx = jnp.max(lo, 1, keepdims=True)
        out_ref[...] = lo - mx - jnp.log(
            jnp.sum(jnp.exp(lo - mx), 1, keepdims=True))

    return pl.pallas_call(
        body,
        out_shape=jax.ShapeDtypeStruct((1024, 10), F32),
    )(sums, cnts, g, be, w1, b1, w2, b2)


def kernel(x, edge_index, batch, eps1, W1a, b1a, W1b, b1b, g1, be1,
           eps2, W2a, b2a, W2b, b2b, g2, be2,
           eps3, W3a, b3a, W3b, b3b, g3, be3,
           fc1W, fc1b, fc2W, fc2b):
    src = edge_index[0].astype(jnp.int32)
    dst = edge_index[1].astype(jnp.int32)
    pad = jnp.arange(EPAD - E, dtype=jnp.int32) % 1024
    src2d = jnp.concatenate([src, pad]).reshape(NGRP, GRP)
    dst2d = jnp.concatenate([dst, N + pad]).reshape(NGRP, GRP)
    xb = jnp.pad(x, ((0, NPAD - N), (0, 13))).reshape(NPAD // 8, 128)
    b2d = jnp.concatenate(
        [batch.astype(jnp.int32),
         jnp.full((NPAD - N,), 1024, jnp.int32)]).reshape(PG, PGRP)
    r1 = lambda a: a.reshape(1, -1)
    e1, e2, e3 = (jnp.reshape(e, (1, 1)) for e in (eps1, eps2, eps3))
    W1ap = jnp.pad(W1a, ((0, 13), (0, 0)))
    i4 = jnp.eye(4, dtype=F32)
    i8 = jnp.eye(8, dtype=F32)

    def kq(W):
        return tuple(jnp.kron(i4, W[a:a + 32, b:b + 32])
                     for b in (0, 32) for a in (0, 32))

    def tb4(b):
        return (jnp.tile(b[:32].reshape(1, 32), (1, 4)),
                jnp.tile(b[32:].reshape(1, 32), (1, 4)))

    p1 = _sc_agg(xb.reshape(NPAD, 16), src2d, dst2d, 16)
    u0, u1 = _tc_l1a(xb, p1.reshape(2, NPAD // 8, 128), e1,
                     jnp.kron(i8, W1ap[:, :32]), jnp.kron(i8, W1ap[:, 32:]),
                     jnp.tile(b1a[:32].reshape(1, 32), (1, 8)),
                     jnp.tile(b1a[32:].reshape(1, 32), (1, 8)))
    h0, h1, ss, sq = _tc_mlp(u0.reshape(NPAD // 4, 128),
                             u1.reshape(NPAD // 4, 128),
                             None, None, None, None, None,
                             kq(W1b), tb4(b1b), split=True)
    y0, y1 = _tc_bn(h0, h1, ss, sq, r1(g1), r1(be1))

    pA = _sc_agg(y0.reshape(NPAD, 32), src2d, dst2d, 32)
    pB = _sc_agg(y1.reshape(NPAD, 32), src2d, dst2d, 32)
    h0, h1, ss, sq = _tc_mlp(y0, y1, pA, pB, e2, kq(W2a), tb4(b2a),
                             kq(W2b), tb4(b2b), split=True)
    y0, y1 = _tc_bn(h0, h1, ss, sq, r1(g2), r1(be2))

    pA = _sc_agg(y0.reshape(NPAD, 32), src2d, dst2d, 32)
    pB = _sc_agg(y1.reshape(NPAD, 32), src2d, dst2d, 32)
    h3 = _tc_mlp(y0, y1, pA, pB, e3, kq(W3a), tb4(b3a),
                 kq(W3b), tb4(b3b), split=False)

    sums, cnts = _sc_pool(h3.reshape(NPAD, 64), b2d)
    return _tc_head(sums, cnts, r1(g3), r1(be3), fc1W, r1(fc1b), fc2W,
                    r1(fc2b))


# speedup vs baseline: 1.3516x; 1.0115x over previous
import jax
import jax.numpy as jnp
from jax import lax
from jax.experimental import pallas as pl
from jax.experimental.pallas import tpu as pltpu
from jax.experimental.pallas import tpu_sc as plsc

N = 50000
NPAD = 53248
E = 800000
EPAD = 819200
GRP = 128
NGRP = EPAD // GRP
NW = 32
GPW = NGRP // NW
IC = 40
NBLK = GPW // IC
NACC = 51200
ROWS_PT = NACC // 16
NZDMA = 8
ZR = 400
GACC = 1056
PGRP = 128
PG = NPAD // PGRP
PGPW = PG // NW
RB = 1024
GRIDN = NPAD // RB
F32 = jnp.float32


def _sc_agg(table, src2d, dst2d, F):
    mesh = plsc.VectorSubcoreMesh(core_axis_name="c", subcore_axis_name="s")
    fl = F // 16
    cw = 4 if F == 16 else 2
    nch = IC // cw

    def body(table_ref, src_ref, dst_ref, out_ref, sidx, didx, rows,
             acc, sg0, sg1, ss0, ss1):
        c = lax.axis_index("c")
        s = lax.axis_index("s")
        w = c * 16 + s
        zv = jnp.zeros((16,), F32)
        semg = (sg0, sg1)
        sems = (ss0, ss1)

        def zinit(i, _):
            rows[i // fl, pl.ds((i % fl) * 16, 16)] = zv
            return 0
        lax.fori_loop(0, 2 * GRP * fl, zinit, 0)

        def zdma(q, _):
            pltpu.sync_copy(rows.at[pl.ds(0, ZR // 2)],
                            acc.at[pl.ds(s * ROWS_PT + q * (ZR // 2),
                                         ZR // 2)])
            return 0
        lax.fori_loop(0, 2 * NZDMA, zdma, 0)
        plsc.subcore_barrier()

        def gath(pc, j):
            par = pc % 2
            return pltpu.async_copy(
                table_ref.at[sidx.at[cw * pc + j]],
                rows.at[pl.ds((par * cw + j) * GRP, GRP)], semg[par])

        def scat(pc, j):
            par = pc % 2
            return pltpu.async_copy(
                rows.at[pl.ds((par * cw + j) * GRP, GRP)],
                acc.at[didx.at[cw * pc + j]], sems[par], add=True)

        def block(ib, _):
            g0 = w * GPW + ib * IC
            pltpu.sync_copy(src_ref.at[pl.ds(g0, IC)], sidx)
            pltpu.sync_copy(dst_ref.at[pl.ds(g0, IC)], didx)
            gd = {0: [gath(0, j) for j in range(cw)]}
            sd = {}
            for pc in range(nch):
                for d in gd.pop(pc):
                    d.wait()
                if pc >= 1:
                    for d in sd.pop(pc - 1):
                        d.wait()
                if pc < nch - 1:
                    gd[pc + 1] = [gath(pc + 1, j) for j in range(cw)]
                sd[pc] = [scat(pc, j) for j in range(cw)]
            for d in sd.pop(nch - 1):
                d.wait()
            return 0
        lax.fori_loop(0, NBLK, block, 0)
        plsc.subcore_barrier()
        pltpu.sync_copy(acc.at[pl.ds(s * ROWS_PT, ROWS_PT)],
                        out_ref.at[c, pl.ds(s * ROWS_PT, ROWS_PT)])

    k = pl.kernel(
        body,
        out_type=jax.ShapeDtypeStruct((2, NPAD, F), F32),
        mesh=mesh,
        compiler_params=pltpu.CompilerParams(use_tc_tiling_on_sc=False),
        scratch_types=[
            pltpu.VMEM((IC, GRP), jnp.int32),
            pltpu.VMEM((IC, GRP), jnp.int32),
            pltpu.VMEM((2 * cw * GRP, F), F32),
            pltpu.VMEM_SHARED((NACC, F), F32),
            pltpu.SemaphoreType.DMA,
            pltpu.SemaphoreType.DMA,
            pltpu.SemaphoreType.DMA,
            pltpu.SemaphoreType.DMA,
        ],
    )
    return k(table, src2d, dst2d)


def _sc_pool(h3, h3b, batch2d):
    mesh = plsc.VectorSubcoreMesh(core_axis_name="c", subcore_axis_name="s")
    GR = GACC // 16

    def body(h0_ref, h1_ref, b_ref, outs0_ref, outs1_ref, outc_ref,
             rows0, rows1, bidx, ones_v, zbs, zbc, accs0, accs1, accc):
        c = lax.axis_index("c")
        s = lax.axis_index("s")
        w = c * 16 + s
        zv = jnp.zeros((16,), F32)
        ov = jnp.full((16,), 1.0, F32)

        def oinit(i, _):
            ones_v[i, pl.ds(0, 16)] = ov
            return 0
        lax.fori_loop(0, PGRP, oinit, 0)

        def zsinit(i, _):
            zbs[i // 2, pl.ds((i % 2) * 16, 16)] = zv
            return 0
        lax.fori_loop(0, GR * 2, zsinit, 0)

        def zcinit(i, _):
            zbc[i, pl.ds(0, 16)] = zv
            return 0
        lax.fori_loop(0, GR, zcinit, 0)
        pltpu.sync_copy(zbs, accs0.at[pl.ds(s * GR, GR)])
        pltpu.sync_copy(zbs, accs1.at[pl.ds(s * GR, GR)])
        pltpu.sync_copy(zbc, accc.at[pl.ds(s * GR, GR)])
        plsc.subcore_barrier()

        def grp(k, _):
            g = w * PGPW + k
            pltpu.sync_copy(b_ref.at[pl.ds(g, 1)], bidx)
            pltpu.sync_copy(h0_ref.at[pl.ds(g * PGRP, PGRP)], rows0)
            pltpu.sync_copy(h1_ref.at[pl.ds(g * PGRP, PGRP)], rows1)
            pltpu.sync_copy(rows0, accs0.at[bidx.at[0]], add=True)
            pltpu.sync_copy(rows1, accs1.at[bidx.at[0]], add=True)
            pltpu.sync_copy(ones_v, accc.at[bidx.at[0]], add=True)
            return 0
        lax.fori_loop(0, PGPW, grp, 0)
        plsc.subcore_barrier()

        @pl.when(s == 0)
        def _():
            pltpu.sync_copy(accs0, outs0_ref.at[c])
            pltpu.sync_copy(accs1, outs1_ref.at[c])
            pltpu.sync_copy(accc, outc_ref.at[c])

    k = pl.kernel(
        body,
        out_type=(jax.ShapeDtypeStruct((2, GACC, 32), F32),
                  jax.ShapeDtypeStruct((2, GACC, 32), F32),
                  jax.ShapeDtypeStruct((2, GACC, 16), F32)),
        mesh=mesh,
        compiler_params=pltpu.CompilerParams(use_tc_tiling_on_sc=False),
        scratch_types=[
            pltpu.VMEM((PGRP, 32), F32),
            pltpu.VMEM((PGRP, 32), F32),
            pltpu.VMEM((1, PGRP), jnp.int32),
            pltpu.VMEM((PGRP, 16), F32),
            pltpu.VMEM((GR, 32), F32),
            pltpu.VMEM((GR, 16), F32),
            pltpu.VMEM_SHARED((GACC, 32), F32),
            pltpu.VMEM_SHARED((GACC, 32), F32),
            pltpu.VMEM_SHARED((GACC, 16), F32),
        ],
    )
    return k(h3, h3b, batch2d)


def _full(shape):
    return pl.BlockSpec(shape, lambda i: (0,) * len(shape))


def _bmask(i):
    r = lax.broadcasted_iota(jnp.int32, (RB // 4, 128), 0)
    l = lax.broadcasted_iota(jnp.int32, (RB // 4, 128), 1)
    return (i * RB + 4 * r + l // 32) < N


def _fold4(q):
    return (q[:, 0:32] + q[:, 32:64] + q[:, 64:96] + q[:, 96:128])


def _dot(a, b):
    return jnp.dot(a, b, preferred_element_type=F32)


def _tc_l1a(xb, p1b, eps, K0, K1, ba0t, ba1t):
    def body(x_ref, p_ref, e_ref, k0_ref, k1_ref, b0_ref, b1_ref,
             u0_ref, u1_ref):
        z = (1.0 + e_ref[0, 0]) * x_ref[...] + p_ref[0] + p_ref[1]
        u0_ref[...] = jnp.maximum(_dot(z, k0_ref[...]) + b0_ref[...], 0.0)
        u1_ref[...] = jnp.maximum(_dot(z, k1_ref[...]) + b1_ref[...], 0.0)

    return pl.pallas_call(
        body, grid=(GRIDN,),
        in_specs=[pl.BlockSpec((RB // 8, 128), lambda i: (i, 0)),
                  pl.BlockSpec((2, RB // 8, 128), lambda i: (0, i, 0)),
                  _full((1, 1)), _full((128, 256)), _full((128, 256)),
                  _full((1, 256)), _full((1, 256))],
        out_specs=[pl.BlockSpec((RB // 8, 256), lambda i: (i, 0)),
                   pl.BlockSpec((RB // 8, 256), lambda i: (i, 0))],
        out_shape=[jax.ShapeDtypeStruct((NPAD // 8, 256), F32),
                   jax.ShapeDtypeStruct((NPAD // 8, 256), F32)],
    )(xb, p1b, eps, K0, K1, ba0t, ba1t)


def _tc_mlp(y0, y1, pA, pB, eps, KA, baT, KB, bbT, *, split):
    two_stage = pA is not None

    def body(*refs):
        i = pl.program_id(0)
        if two_stage:
            (y0_ref, y1_ref, pa_ref, pb_ref, e_ref, ka00, ka10, ka01, ka11,
             ba0, ba1, kb00, kb10, kb01, kb11, bb0, bb1, *outs) = refs
            s = 1.0 + e_ref[0, 0]
            z0 = s * y0_ref[...] + pa_ref[0] + pa_ref[1]
            z1 = s * y1_ref[...] + pb_ref[0] + pb_ref[1]
            u0 = jnp.maximum(_dot(z0, ka00[...]) + _dot(z1, ka10[...])
                             + ba0[...], 0.0)
            u1 = jnp.maximum(_dot(z0, ka01[...]) + _dot(z1, ka11[...])
                             + ba1[...], 0.0)
        else:
            (y0_ref, y1_ref, kb00, kb10, kb01, kb11, bb0, bb1, *outs) = refs
            u0 = y0_ref[...]
            u1 = y1_ref[...]
        v0 = jnp.maximum(_dot(u0, kb00[...]) + _dot(u1, kb10[...])
                         + bb0[...], 0.0)
        v1 = jnp.maximum(_dot(u0, kb01[...]) + _dot(u1, kb11[...])
                         + bb1[...], 0.0)
        m = _bmask(i)
        v0 = jnp.where(m, v0, 0.0)
        v1 = jnp.where(m, v1, 0.0)
        if split:
            h0_ref, h1_ref, ssum_ref, ssq_ref = outs

            @pl.when(i == 0)
            def _():
                ssum_ref[...] = jnp.zeros_like(ssum_ref)
                ssq_ref[...] = jnp.zeros_like(ssq_ref)
            h0_ref[...] = v0
            h1_ref[...] = v1
            ssum_ref[...] += jnp.concatenate(
                [_fold4(jnp.sum(v0, 0, keepdims=True)),
                 _fold4(jnp.sum(v1, 0, keepdims=True))], axis=1)
            ssq_ref[...] += jnp.concatenate(
                [_fold4(jnp.sum(v0 * v0, 0, keepdims=True)),
                 _fold4(jnp.sum(v1 * v1, 0, keepdims=True))], axis=1)
        else:
            outs[0][...] = v0
            outs[1][...] = v1

    b32 = pl.BlockSpec((RB // 4, 128), lambda i: (i, 0))
    in_specs = [b32, b32]
    args = [y0, y1]
    if two_stage:
        in_specs += [pl.BlockSpec((2, RB // 4, 128), lambda i: (0, i, 0))] * 2
        in_specs += [_full((1, 1))] + [_full((128, 128))] * 4 \
            + [_full((1, 128))] * 2
        args += [pA.reshape(2, NPAD // 4, 128), pB.reshape(2, NPAD // 4, 128),
                 eps, *KA, *baT]
    in_specs += [_full((128, 128))] * 4 + [_full((1, 128))] * 2
    args += [*KB, *bbT]
    if split:
        out_specs = [b32, b32, _full((1, 64)), _full((1, 64))]
        out_shape = [jax.ShapeDtypeStruct((NPAD // 4, 128), F32),
                     jax.ShapeDtypeStruct((NPAD // 4, 128), F32),
                     jax.ShapeDtypeStruct((1, 64), F32),
                     jax.ShapeDtypeStruct((1, 64), F32)]
    else:
        out_specs = [b32, b32]
        out_shape = [jax.ShapeDtypeStruct((NPAD // 4, 128), F32),
                     jax.ShapeDtypeStruct((NPAD // 4, 128), F32)]

    return pl.pallas_call(body, grid=(GRIDN,), in_specs=in_specs,
                          out_specs=out_specs, out_shape=out_shape)(*args)


def _tc_bn(h0, h1, ssum, ssq, g, be):
    def body(h0_ref, h1_ref, ssum_ref, ssq_ref, g_ref, be_ref, y0_ref, y1_ref):
        m = ssum_ref[...] / N
        v = ssq_ref[...] / N - m * m
        a = g_ref[...] * lax.rsqrt(v + 1e-5)
        sh = be_ref[...] - m * a
        a0 = jnp.concatenate([a[:, :32]] * 4, axis=1)
        a1 = jnp.concatenate([a[:, 32:]] * 4, axis=1)
        s0 = jnp.concatenate([sh[:, :32]] * 4, axis=1)
        s1 = jnp.concatenate([sh[:, 32:]] * 4, axis=1)
        y0_ref[...] = h0_ref[...] * a0 + s0
        y1_ref[...] = h1_ref[...] * a1 + s1

    return pl.pallas_call(
        body, grid=(GRIDN,),
        in_specs=[pl.BlockSpec((RB // 4, 128), lambda i: (i, 0)),
                  pl.BlockSpec((RB // 4, 128), lambda i: (i, 0)),
                  _full((1, 64)), _full((1, 64)), _full((1, 64)),
                  _full((1, 64))],
        out_specs=[pl.BlockSpec((RB // 4, 128), lambda i: (i, 0)),
                   pl.BlockSpec((RB // 4, 128), lambda i: (i, 0))],
        out_shape=[jax.ShapeDtypeStruct((NPAD // 4, 128), F32),
                   jax.ShapeDtypeStruct((NPAD // 4, 128), F32)],
    )(h0, h1, ssum, ssq, g, be)


def _tc_head(sums0, sums1, cnts, g, be, w1, b1, w2, b2):
    def body(s0_ref, s1_ref, c_ref, g_ref, be_ref, w1_ref, b1_ref, w2_ref,
             b2_ref, out_ref):
        s = jnp.concatenate([(s0_ref[0] + s0_ref[1])[:1024],
                             (s1_ref[0] + s1_ref[1])[:1024]], axis=1)
        cnt = (c_ref[0] + c_ref[1])[:1024, 0:1]
        pooled = s / jnp.maximum(cnt, 1.0)
        m = jnp.mean(pooled, 0, keepdims=True)
        v = jnp.mean(pooled * pooled, 0, keepdims=True) - m * m
        pooled = (pooled - m) * lax.rsqrt(v + 1e-5) * g_ref[...] + be_ref[...]
        t = jnp.dot(pooled, w1_ref[...], preferred_element_type=F32) \
            + b1_ref[...]
        t = jnp.where(t > 0, t, jnp.exp(jnp.minimum(t, 0.0)) - 1.0)
        lo = jnp.dot(t, w2_ref[...], preferred_element_type=F32) + b2_ref[...]
        mx = jnp.max(lo, 1, keepdims=True)
        out_ref[...] = lo - mx - jnp.log(
            jnp.sum(jnp.exp(lo - mx), 1, keepdims=True))

    return pl.pallas_call(
        body,
        out_shape=jax.ShapeDtypeStruct((1024, 10), F32),
    )(sums0, sums1, cnts, g, be, w1, b1, w2, b2)


def kernel(x, edge_index, batch, eps1, W1a, b1a, W1b, b1b, g1, be1,
           eps2, W2a, b2a, W2b, b2b, g2, be2,
           eps3, W3a, b3a, W3b, b3b, g3, be3,
           fc1W, fc1b, fc2W, fc2b):
    src = edge_index[0].astype(jnp.int32)
    dst = edge_index[1].astype(jnp.int32)
    pad = jnp.arange(EPAD - E, dtype=jnp.int32) % 1024
    src2d = jnp.concatenate([src, pad]).reshape(NGRP, GRP)
    dst2d = jnp.concatenate([dst, N + pad]).reshape(NGRP, GRP)
    xb = jnp.pad(x, ((0, NPAD - N), (0, 13))).reshape(NPAD // 8, 128)
    b2d = jnp.concatenate(
        [batch.astype(jnp.int32),
         jnp.full((NPAD - N,), 1024, jnp.int32)]).reshape(PG, PGRP)
    r1 = lambda a: a.reshape(1, -1)
    e1, e2, e3 = (jnp.reshape(e, (1, 1)) for e in (eps1, eps2, eps3))
    W1ap = jnp.pad(W1a, ((0, 13), (0, 0)))
    i4 = jnp.eye(4, dtype=F32)
    i8 = jnp.eye(8, dtype=F32)

    def kq(W):
        return tuple(jnp.kron(i4, W[a:a + 32, b:b + 32])
                     for b in (0, 32) for a in (0, 32))

    def tb4(b):
        return (jnp.tile(b[:32].reshape(1, 32), (1, 4)),
                jnp.tile(b[32:].reshape(1, 32), (1, 4)))

    p1 = _sc_agg(xb.reshape(NPAD, 16), src2d, dst2d, 16)
    u0, u1 = _tc_l1a(xb, p1.reshape(2, NPAD // 8, 128), e1,
                     jnp.kron(i8, W1ap[:, :32]), jnp.kron(i8, W1ap[:, 32:]),
                     jnp.tile(b1a[:32].reshape(1, 32), (1, 8)),
                     jnp.tile(b1a[32:].reshape(1, 32), (1, 8)))
    h0, h1, ss, sq = _tc_mlp(u0.reshape(NPAD // 4, 128),
                             u1.reshape(NPAD // 4, 128),
                             None, None, None, None, None,
                             kq(W1b), tb4(b1b), split=True)
    y0, y1 = _tc_bn(h0, h1, ss, sq, r1(g1), r1(be1))

    pA = _sc_agg(y0.reshape(NPAD, 32), src2d, dst2d, 32)
    pB = _sc_agg(y1.reshape(NPAD, 32), src2d, dst2d, 32)
    h0, h1, ss, sq = _tc_mlp(y0, y1, pA, pB, e2, kq(W2a), tb4(b2a),
                             kq(W2b), tb4(b2b), split=True)
    y0, y1 = _tc_bn(h0, h1, ss, sq, r1(g2), r1(be2))

    pA = _sc_agg(y0.reshape(NPAD, 32), src2d, dst2d, 32)
    pB = _sc_agg(y1.reshape(NPAD, 32), src2d, dst2d, 32)
    h30, h31 = _tc_mlp(y0, y1, pA, pB, e3, kq(W3a), tb4(b3a),
                       kq(W3b), tb4(b3b), split=False)

    sums0, sums1, cnts = _sc_pool(h30.reshape(NPAD, 32),
                                  h31.reshape(NPAD, 32), b2d)
    return _tc_head(sums0, sums1, cnts, r1(g3), r1(be3), fc1W, r1(fc1b),
                    fc2W, r1(fc2b))


# speedup vs baseline: 1.3669x; 1.0114x over previous
import jax
import jax.numpy as jnp
from jax import lax
from jax.experimental import pallas as pl
from jax.experimental.pallas import tpu as pltpu
from jax.experimental.pallas import tpu_sc as plsc

N = 50000
NPAD = 53248
E = 800000
EPAD = 819200
GRP = 128
NGRP = EPAD // GRP
NW = 32
GPW = NGRP // NW
IC = 40
NBLK = GPW // IC
NACC = 51200
ROWS_PT = NACC // 16
NZDMA = 8
ZR = 400
GACC = 1056
PGRP = 128
PG = NPAD // PGRP
PGPW = PG // NW
RB = 1024
GRIDN = NPAD // RB
F32 = jnp.float32


def _sc_agg(table, src2d, dst2d, F):
    mesh = plsc.VectorSubcoreMesh(core_axis_name="c", subcore_axis_name="s")
    fl = F // 16
    cw = 8 if F == 16 else 2
    nch = IC // cw

    def body(table_ref, src_ref, dst_ref, out_ref, sidx, didx, rows,
             acc, sg0, sg1, ss0, ss1):
        c = lax.axis_index("c")
        s = lax.axis_index("s")
        w = c * 16 + s
        zv = jnp.zeros((16,), F32)
        semg = (sg0, sg1)
        sems = (ss0, ss1)

        def zinit(i, _):
            rows[i // fl, pl.ds((i % fl) * 16, 16)] = zv
            return 0
        lax.fori_loop(0, 2 * GRP * fl, zinit, 0)

        def zdma(q, _):
            pltpu.sync_copy(rows.at[pl.ds(0, ZR // 2)],
                            acc.at[pl.ds(s * ROWS_PT + q * (ZR // 2),
                                         ZR // 2)])
            return 0
        lax.fori_loop(0, 2 * NZDMA, zdma, 0)
        plsc.subcore_barrier()

        def gath(pc, j):
            par = pc % 2
            return pltpu.async_copy(
                table_ref.at[sidx.at[cw * pc + j]],
                rows.at[pl.ds((par * cw + j) * GRP, GRP)], semg[par])

        def scat(pc, j):
            par = pc % 2
            return pltpu.async_copy(
                rows.at[pl.ds((par * cw + j) * GRP, GRP)],
                acc.at[didx.at[cw * pc + j]], sems[par], add=True)

        def block(ib, _):
            g0 = w * GPW + ib * IC
            pltpu.sync_copy(src_ref.at[pl.ds(g0, IC)], sidx)
            pltpu.sync_copy(dst_ref.at[pl.ds(g0, IC)], didx)
            gd = {0: [gath(0, j) for j in range(cw)]}
            sd = {}
            for pc in range(nch):
                for d in gd.pop(pc):
                    d.wait()
                if pc >= 1:
                    for d in sd.pop(pc - 1):
                        d.wait()
                if pc < nch - 1:
                    gd[pc + 1] = [gath(pc + 1, j) for j in range(cw)]
                sd[pc] = [scat(pc, j) for j in range(cw)]
            for d in sd.pop(nch - 1):
                d.wait()
            return 0
        lax.fori_loop(0, NBLK, block, 0)
        plsc.subcore_barrier()
        pltpu.sync_copy(acc.at[pl.ds(s * ROWS_PT, ROWS_PT)],
                        out_ref.at[c, pl.ds(s * ROWS_PT, ROWS_PT)])

    k = pl.kernel(
        body,
        out_type=jax.ShapeDtypeStruct((2, NPAD, F), F32),
        mesh=mesh,
        compiler_params=pltpu.CompilerParams(use_tc_tiling_on_sc=False),
        scratch_types=[
            pltpu.VMEM((IC, GRP), jnp.int32),
            pltpu.VMEM((IC, GRP), jnp.int32),
            pltpu.VMEM((2 * cw * GRP, F), F32),
            pltpu.VMEM_SHARED((NACC, F), F32),
            pltpu.SemaphoreType.DMA,
            pltpu.SemaphoreType.DMA,
            pltpu.SemaphoreType.DMA,
            pltpu.SemaphoreType.DMA,
        ],
    )
    return k(table, src2d, dst2d)


def _sc_pool(h3, h3b, batch2d):
    mesh = plsc.VectorSubcoreMesh(core_axis_name="c", subcore_axis_name="s")
    GR = GACC // 16

    def body(h0_ref, h1_ref, b_ref, outs0_ref, outs1_ref, outc_ref,
             rows0, rows1, bidx, ones_v, zbs, zbc, accs0, accs1, accc):
        c = lax.axis_index("c")
        s = lax.axis_index("s")
        w = c * 16 + s
        zv = jnp.zeros((16,), F32)
        ov = jnp.full((16,), 1.0, F32)

        def oinit(i, _):
            ones_v[i, pl.ds(0, 16)] = ov
            return 0
        lax.fori_loop(0, PGRP, oinit, 0)

        def zsinit(i, _):
            zbs[i // 2, pl.ds((i % 2) * 16, 16)] = zv
            return 0
        lax.fori_loop(0, GR * 2, zsinit, 0)

        def zcinit(i, _):
            zbc[i, pl.ds(0, 16)] = zv
            return 0
        lax.fori_loop(0, GR, zcinit, 0)
        pltpu.sync_copy(zbs, accs0.at[pl.ds(s * GR, GR)])
        pltpu.sync_copy(zbs, accs1.at[pl.ds(s * GR, GR)])
        pltpu.sync_copy(zbc, accc.at[pl.ds(s * GR, GR)])
        plsc.subcore_barrier()

        def grp(k, _):
            g = w * PGPW + k
            pltpu.sync_copy(b_ref.at[pl.ds(g, 1)], bidx)
            pltpu.sync_copy(h0_ref.at[pl.ds(g * PGRP, PGRP)], rows0)
            pltpu.sync_copy(h1_ref.at[pl.ds(g * PGRP, PGRP)], rows1)
            pltpu.sync_copy(rows0, accs0.at[bidx.at[0]], add=True)
            pltpu.sync_copy(rows1, accs1.at[bidx.at[0]], add=True)
            pltpu.sync_copy(ones_v, accc.at[bidx.at[0]], add=True)
            return 0
        lax.fori_loop(0, PGPW, grp, 0)
        plsc.subcore_barrier()

        @pl.when(s == 0)
        def _():
            pltpu.sync_copy(accs0, outs0_ref.at[c])
            pltpu.sync_copy(accs1, outs1_ref.at[c])
            pltpu.sync_copy(accc, outc_ref.at[c])

    k = pl.kernel(
        body,
        out_type=(jax.ShapeDtypeStruct((2, GACC, 32), F32),
                  jax.ShapeDtypeStruct((2, GACC, 32), F32),
                  jax.ShapeDtypeStruct((2, GACC, 16), F32)),
        mesh=mesh,
        compiler_params=pltpu.CompilerParams(use_tc_tiling_on_sc=False),
        scratch_types=[
            pltpu.VMEM((PGRP, 32), F32),
            pltpu.VMEM((PGRP, 32), F32),
            pltpu.VMEM((1, PGRP), jnp.int32),
            pltpu.VMEM((PGRP, 16), F32),
            pltpu.VMEM((GR, 32), F32),
            pltpu.VMEM((GR, 16), F32),
            pltpu.VMEM_SHARED((GACC, 32), F32),
            pltpu.VMEM_SHARED((GACC, 32), F32),
            pltpu.VMEM_SHARED((GACC, 16), F32),
        ],
    )
    return k(h3, h3b, batch2d)


def _full(shape):
    return pl.BlockSpec(shape, lambda i: (0,) * len(shape))


def _bmask(i):
    r = lax.broadcasted_iota(jnp.int32, (RB // 4, 128), 0)
    l = lax.broadcasted_iota(jnp.int32, (RB // 4, 128), 1)
    return (i * RB + 4 * r + l // 32) < N


def _fold4(q):
    return (q[:, 0:32] + q[:, 32:64] + q[:, 64:96] + q[:, 96:128])


def _dot(a, b):
    return jnp.dot(a, b, preferred_element_type=F32)


def _tc_l1a(xb, p1b, eps, K0, K1, ba0t, ba1t):
    def body(x_ref, p_ref, e_ref, k0_ref, k1_ref, b0_ref, b1_ref,
             u0_ref, u1_ref):
        z = (1.0 + e_ref[0, 0]) * x_ref[...] + p_ref[0] + p_ref[1]
        u0_ref[...] = jnp.maximum(_dot(z, k0_ref[...]) + b0_ref[...], 0.0)
        u1_ref[...] = jnp.maximum(_dot(z, k1_ref[...]) + b1_ref[...], 0.0)

    return pl.pallas_call(
        body, grid=(GRIDN,),
        in_specs=[pl.BlockSpec((RB // 8, 128), lambda i: (i, 0)),
                  pl.BlockSpec((2, RB // 8, 128), lambda i: (0, i, 0)),
                  _full((1, 1)), _full((128, 256)), _full((128, 256)),
                  _full((1, 256)), _full((1, 256))],
        out_specs=[pl.BlockSpec((RB // 8, 256), lambda i: (i, 0)),
                   pl.BlockSpec((RB // 8, 256), lambda i: (i, 0))],
        out_shape=[jax.ShapeDtypeStruct((NPAD // 8, 256), F32),
                   jax.ShapeDtypeStruct((NPAD // 8, 256), F32)],
    )(xb, p1b, eps, K0, K1, ba0t, ba1t)


def _tc_mlp(y0, y1, pA, pB, eps, KA, baT, KB, bbT, *, split):
    two_stage = pA is not None

    def body(*refs):
        i = pl.program_id(0)
        if two_stage:
            (y0_ref, y1_ref, pa_ref, pb_ref, e_ref, ka00, ka10, ka01, ka11,
             ba0, ba1, kb00, kb10, kb01, kb11, bb0, bb1, *outs) = refs
            s = 1.0 + e_ref[0, 0]
            z0 = s * y0_ref[...] + pa_ref[0] + pa_ref[1]
            z1 = s * y1_ref[...] + pb_ref[0] + pb_ref[1]
            u0 = jnp.maximum(_dot(z0, ka00[...]) + _dot(z1, ka10[...])
                             + ba0[...], 0.0)
            u1 = jnp.maximum(_dot(z0, ka01[...]) + _dot(z1, ka11[...])
                             + ba1[...], 0.0)
        else:
            (y0_ref, y1_ref, kb00, kb10, kb01, kb11, bb0, bb1, *outs) = refs
            u0 = y0_ref[...]
            u1 = y1_ref[...]
        v0 = jnp.maximum(_dot(u0, kb00[...]) + _dot(u1, kb10[...])
                         + bb0[...], 0.0)
        v1 = jnp.maximum(_dot(u0, kb01[...]) + _dot(u1, kb11[...])
                         + bb1[...], 0.0)
        m = _bmask(i)
        v0 = jnp.where(m, v0, 0.0)
        v1 = jnp.where(m, v1, 0.0)
        if split:
            h0_ref, h1_ref, ssum_ref, ssq_ref = outs

            @pl.when(i == 0)
            def _():
                ssum_ref[...] = jnp.zeros_like(ssum_ref)
                ssq_ref[...] = jnp.zeros_like(ssq_ref)
            h0_ref[...] = v0
            h1_ref[...] = v1
            ssum_ref[...] += jnp.concatenate(
                [_fold4(jnp.sum(v0, 0, keepdims=True)),
                 _fold4(jnp.sum(v1, 0, keepdims=True))], axis=1)
            ssq_ref[...] += jnp.concatenate(
                [_fold4(jnp.sum(v0 * v0, 0, keepdims=True)),
                 _fold4(jnp.sum(v1 * v1, 0, keepdims=True))], axis=1)
        else:
            outs[0][...] = v0
            outs[1][...] = v1

    b32 = pl.BlockSpec((RB // 4, 128), lambda i: (i, 0))
    in_specs = [b32, b32]
    args = [y0, y1]
    if two_stage:
        in_specs += [pl.BlockSpec((2, RB // 4, 128), lambda i: (0, i, 0))] * 2
        in_specs += [_full((1, 1))] + [_full((128, 128))] * 4 \
            + [_full((1, 128))] * 2
        args += [pA.reshape(2, NPAD // 4, 128), pB.reshape(2, NPAD // 4, 128),
                 eps, *KA, *baT]
    in_specs += [_full((128, 128))] * 4 + [_full((1, 128))] * 2
    args += [*KB, *bbT]
    if split:
        out_specs = [b32, b32, _full((1, 64)), _full((1, 64))]
        out_shape = [jax.ShapeDtypeStruct((NPAD // 4, 128), F32),
                     jax.ShapeDtypeStruct((NPAD // 4, 128), F32),
                     jax.ShapeDtypeStruct((1, 64), F32),
                     jax.ShapeDtypeStruct((1, 64), F32)]
    else:
        out_specs = [b32, b32]
        out_shape = [jax.ShapeDtypeStruct((NPAD // 4, 128), F32),
                     jax.ShapeDtypeStruct((NPAD // 4, 128), F32)]

    return pl.pallas_call(body, grid=(GRIDN,), in_specs=in_specs,
                          out_specs=out_specs, out_shape=out_shape)(*args)


def _tc_bn(h0, h1, ssum, ssq, g, be):
    def body(h0_ref, h1_ref, ssum_ref, ssq_ref, g_ref, be_ref, y0_ref, y1_ref):
        m = ssum_ref[...] / N
        v = ssq_ref[...] / N - m * m
        a = g_ref[...] * lax.rsqrt(v + 1e-5)
        sh = be_ref[...] - m * a
        a0 = jnp.concatenate([a[:, :32]] * 4, axis=1)
        a1 = jnp.concatenate([a[:, 32:]] * 4, axis=1)
        s0 = jnp.concatenate([sh[:, :32]] * 4, axis=1)
        s1 = jnp.concatenate([sh[:, 32:]] * 4, axis=1)
        y0_ref[...] = h0_ref[...] * a0 + s0
        y1_ref[...] = h1_ref[...] * a1 + s1

    return pl.pallas_call(
        body, grid=(GRIDN,),
        in_specs=[pl.BlockSpec((RB // 4, 128), lambda i: (i, 0)),
                  pl.BlockSpec((RB // 4, 128), lambda i: (i, 0)),
                  _full((1, 64)), _full((1, 64)), _full((1, 64)),
                  _full((1, 64))],
        out_specs=[pl.BlockSpec((RB // 4, 128), lambda i: (i, 0)),
                   pl.BlockSpec((RB // 4, 128), lambda i: (i, 0))],
        out_shape=[jax.ShapeDtypeStruct((NPAD // 4, 128), F32),
                   jax.ShapeDtypeStruct((NPAD // 4, 128), F32)],
    )(h0, h1, ssum, ssq, g, be)


def _tc_head(sums0, sums1, cnts, g, be, w1, b1, w2, b2):
    def body(s0_ref, s1_ref, c_ref, g_ref, be_ref, w1_ref, b1_ref, w2_ref,
             b2_ref, out_ref):
        s = jnp.concatenate([(s0_ref[0] + s0_ref[1])[:1024],
                             (s1_ref[0] + s1_ref[1])[:1024]], axis=1)
        cnt = (c_ref[0] + c_ref[1])[:1024, 0:1]
        pooled = s / jnp.maximum(cnt, 1.0)
        m = jnp.mean(pooled, 0, keepdims=True)
        v = jnp.mean(pooled * pooled, 0, keepdims=True) - m * m
        pooled = (pooled - m) * lax.rsqrt(v + 1e-5) * g_ref[...] + be_ref[...]
        t = jnp.dot(pooled, w1_ref[...], preferred_element_type=F32) \
            + b1_ref[...]
        t = jnp.where(t > 0, t, jnp.exp(jnp.minimum(t, 0.0)) - 1.0)
        lo = jnp.dot(t, w2_ref[...], preferred_element_type=F32) + b2_ref[...]
        mx = jnp.max(lo, 1, keepdims=True)
        out_ref[...] = lo - mx - jnp.log(
            jnp.sum(jnp.exp(lo - mx), 1, keepdims=True))

    return pl.pallas_call(
        body,
        out_shape=jax.ShapeDtypeStruct((1024, 10), F32),
    )(sums0, sums1, cnts, g, be, w1, b1, w2, b2)


def kernel(x, edge_index, batch, eps1, W1a, b1a, W1b, b1b, g1, be1,
           eps2, W2a, b2a, W2b, b2b, g2, be2,
           eps3, W3a, b3a, W3b, b3b, g3, be3,
           fc1W, fc1b, fc2W, fc2b):
    src = edge_index[0].astype(jnp.int32)
    dst = edge_index[1].astype(jnp.int32)
    pad = jnp.arange(EPAD - E, dtype=jnp.int32) % 1024
    src2d = jnp.concatenate([src, pad]).reshape(NGRP, GRP)
    dst2d = jnp.concatenate([dst, N + pad]).reshape(NGRP, GRP)
    xb = jnp.pad(x, ((0, NPAD - N), (0, 13))).reshape(NPAD // 8, 128)
    b2d = jnp.concatenate(
        [batch.astype(jnp.int32),
         jnp.full((NPAD - N,), 1024, jnp.int32)]).reshape(PG, PGRP)
    r1 = lambda a: a.reshape(1, -1)
    e1, e2, e3 = (jnp.reshape(e, (1, 1)) for e in (eps1, eps2, eps3))
    W1ap = jnp.pad(W1a, ((0, 13), (0, 0)))
    i4 = jnp.eye(4, dtype=F32)
    i8 = jnp.eye(8, dtype=F32)

    def kq(W):
        return tuple(jnp.kron(i4, W[a:a + 32, b:b + 32])
                     for b in (0, 32) for a in (0, 32))

    def tb4(b):
        return (jnp.tile(b[:32].reshape(1, 32), (1, 4)),
                jnp.tile(b[32:].reshape(1, 32), (1, 4)))

    p1 = _sc_agg(xb.reshape(NPAD, 16), src2d, dst2d, 16)
    u0, u1 = _tc_l1a(xb, p1.reshape(2, NPAD // 8, 128), e1,
                     jnp.kron(i8, W1ap[:, :32]), jnp.kron(i8, W1ap[:, 32:]),
                     jnp.tile(b1a[:32].reshape(1, 32), (1, 8)),
                     jnp.tile(b1a[32:].reshape(1, 32), (1, 8)))
    h0, h1, ss, sq = _tc_mlp(u0.reshape(NPAD // 4, 128),
                             u1.reshape(NPAD // 4, 128),
                             None, None, None, None, None,
                             kq(W1b), tb4(b1b), split=True)
    y0, y1 = _tc_bn(h0, h1, ss, sq, r1(g1), r1(be1))

    pA = _sc_agg(y0.reshape(NPAD, 32), src2d, dst2d, 32)
    pB = _sc_agg(y1.reshape(NPAD, 32), src2d, dst2d, 32)
    h0, h1, ss, sq = _tc_mlp(y0, y1, pA, pB, e2, kq(W2a), tb4(b2a),
                             kq(W2b), tb4(b2b), split=True)
    y0, y1 = _tc_bn(h0, h1, ss, sq, r1(g2), r1(be2))

    pA = _sc_agg(y0.reshape(NPAD, 32), src2d, dst2d, 32)
    pB = _sc_agg(y1.reshape(NPAD, 32), src2d, dst2d, 32)
    h30, h31 = _tc_mlp(y0, y1, pA, pB, e3, kq(W3a), tb4(b3a),
                       kq(W3b), tb4(b3b), split=False)

    sums0, sums1, cnts = _sc_pool(h30.reshape(NPAD, 32),
                                  h31.reshape(NPAD, 32), b2d)
    return _tc_head(sums0, sums1, cnts, r1(g3), r1(be3), fc1W, r1(fc1b),
                    fc2W, r1(fc2b))
